# baseline (speedup 1.0000x reference)
import functools

import jax
import jax.numpy as jnp
from jax import lax
from jax.experimental import pallas as pl
from jax.experimental.pallas import tpu as pltpu
from jax.experimental.pallas import tpu_sc as plsc

_NUM_EMBED = 1000000
_EMBED_DIM = 64
_BATCH = 4096
_HIST = 200

_NC = 2
_NS = 16
_NW = _NC * _NS

_ROWS_W = _BATCH // _NW
_NBUF = 2
_NGROUP = _ROWS_W // _NBUF

_LANE_PAD = 2 * _EMBED_DIM
_NSLAB_FULL = _NUM_EMBED // _LANE_PAD
_SLAB_REM = _NUM_EMBED - _NSLAB_FULL * _LANE_PAD
_FMT_ITERS = 246


def _make_format_kernel():
  mesh = plsc.VectorSubcoreMesh(core_axis_name="c", subcore_axis_name="s")

  @functools.partial(
      pl.kernel,
      mesh=mesh,
      compiler_params=pltpu.CompilerParams(
          use_tc_tiling_on_sc=True, needs_layout_passes=False),
      out_type=jax.ShapeDtypeStruct((_NUM_EMBED, _LANE_PAD), jnp.float32),
      scratch_types=[
          pltpu.VMEM((2, _EMBED_DIM, _LANE_PAD), jnp.float32),
          pltpu.VMEM((2, _LANE_PAD, _LANE_PAD), jnp.float32),
          pltpu.VMEM((_EMBED_DIM, _SLAB_REM), jnp.float32),
          pltpu.SemaphoreType.DMA((2,)),
          pltpu.SemaphoreType.DMA((2,)),
      ],
  )
  def k(wt_hbm, wrem_hbm, w128_hbm, slab_v, tslab_v, slab64_v, isem, osem):
    wid = lax.axis_index("s") * _NC + lax.axis_index("c")

    row_ids = [lax.iota(jnp.int32, 16) + 16 * q for q in range(8)]

    def fire_in(j, p):
      pltpu.async_copy(
          wt_hbm.at[:, pl.ds(j * _LANE_PAD, _LANE_PAD)],
          slab_v.at[p], isem.at[p])

    def wait_in(p):
      pltpu.make_async_copy(
          wt_hbm.at[:, pl.ds(0, _LANE_PAD)], slab_v.at[p], isem.at[p]).wait()

    def fire_out(j, p):
      pltpu.async_copy(
          tslab_v.at[p],
          w128_hbm.at[pl.ds(j * _LANE_PAD, _LANE_PAD)],
          osem.at[p])

    def wait_out(p):
      pltpu.make_async_copy(
          tslab_v.at[p],
          w128_hbm.at[pl.ds(0, _LANE_PAD)],
          osem.at[p]).wait()

    def transpose(src, dst, n_q):
      for d in range(_EMBED_DIM):
        col = jnp.full((16,), d, jnp.int32)
        for q in range(n_q):
          v = src[d, pl.ds(16 * q, 16)]
          plsc.store_scatter(dst, [row_ids[q], col], v)

    fire_in(wid, 0)
    fire_in(wid + _NW, 1)

    def body(m, carry):
      for p in (0, 1):
        kk = 2 * m + p
        j = wid + _NW * kk

        @pl.when(j < _NSLAB_FULL)
        def _():
          wait_in(p)

          @pl.when(kk >= 2)
          def _():
            wait_out(p)

          transpose(slab_v.at[p], tslab_v.at[p], 8)
          fire_out(j, p)
          jn = wid + _NW * (kk + 2)

          @pl.when(jn < _NSLAB_FULL)
          def _():
            fire_in(jn, p)

      return carry

    lax.fori_loop(0, _FMT_ITERS // 2, body, 0)

    wait_out(0)
    wait_out(1)

    @pl.when(wid == 0)
    def _():
      base = _NSLAB_FULL * _LANE_PAD
      pltpu.sync_copy(wrem_hbm, slab64_v)
      transpose(slab64_v, tslab_v.at[0], _SLAB_REM // 16)
      pltpu.sync_copy(
          tslab_v.at[0, pl.ds(0, _SLAB_REM)],
          w128_hbm.at[pl.ds(base, _SLAB_REM)])

  return k


def _make_gather_kernel():
  mesh = plsc.VectorSubcoreMesh(core_axis_name="c", subcore_axis_name="s")

  @functools.partial(
      pl.kernel,
      mesh=mesh,
      compiler_params=pltpu.CompilerParams(use_tc_tiling_on_sc=False),
      out_type=jax.ShapeDtypeStruct((_BATCH, _HIST, _LANE_PAD), jnp.float32),
      scratch_types=[
          pltpu.VMEM((_ROWS_W, _HIST), jnp.int32),
          pltpu.VMEM((2, _NBUF, _HIST, _LANE_PAD), jnp.float32),
          pltpu.SemaphoreType.DMA((2, _NBUF)),
          pltpu.SemaphoreType.DMA((2, _NBUF)),
      ],
  )
  def k(idx_hbm, table_hbm, out_hbm, idx_v, rows_v, gsem, ssem):
    wid = lax.axis_index("s") * _NC + lax.axis_index("c")
    base = wid * _ROWS_W

    pltpu.sync_copy(idx_hbm.at[pl.ds(base, _ROWS_W)], idx_v)

    def fire_gather(row, p, b):
      pltpu.async_copy(
          table_hbm.at[idx_v.at[row]], rows_v.at[p, b], gsem.at[p, b])

    def fire_store(row, p, b):
      pltpu.async_copy(
          rows_v.at[p, b, :, pl.ds(0, _EMBED_DIM)],
          out_hbm.at[base + row, :, pl.ds(0, _EMBED_DIM)],
          ssem.at[p, b])

    def wait_gather(p, b):
      pltpu.make_async_copy(
          table_hbm.at[idx_v.at[0]], rows_v.at[p, b], gsem.at[p, b]).wait()

    def wait_store(p, b):
      pltpu.make_async_copy(
          rows_v.at[p, b, :, pl.ds(0, _EMBED_DIM)],
          out_hbm.at[base, :, pl.ds(0, _EMBED_DIM)],
          ssem.at[p, b]).wait()

    for b in range(_NBUF):
      fire_gather(b, 0, b)

    def body(j, carry):
      for p in (0, 1):
        i = 2 * j + p
        q = 1 - p
        for b in range(_NBUF):
          @pl.when(i > 0)
          def _():
            wait_store(q, b)

          @pl.when(i < _NGROUP - 1)
          def _():
            fire_gather((i + 1) * _NBUF + b, q, b)

          wait_gather(p, b)
          fire_store(i * _NBUF + b, p, b)
      return carry

    lax.fori_loop(0, _NGROUP // 2, body, 0)

    for b in range(_NBUF):
      wait_store(1, b)

  return k


_format_call = _make_format_kernel()
_gather_call = _make_gather_kernel()


def kernel(idx, weight):
  wt = weight.T
  w128 = _format_call(wt, wt[:, _NSLAB_FULL * _LANE_PAD:])
  out_pad = _gather_call(idx.astype(jnp.int32), w128)
  return out_pad[:, :, :_EMBED_DIM]


# speedup vs baseline: 1.0062x; 1.0062x over previous
import functools

import jax
import jax.numpy as jnp
from jax import lax
from jax.experimental import pallas as pl
from jax.experimental.pallas import tpu as pltpu
from jax.experimental.pallas import tpu_sc as plsc

_NUM_EMBED = 1000000
_EMBED_DIM = 64
_BATCH = 4096
_HIST = 200

_NC = 2
_NS = 16
_NW = _NC * _NS

_ROWS_W = _BATCH // _NW
_NBUF = 2
_NGROUP = _ROWS_W // _NBUF

_LANE_PAD = 2 * _EMBED_DIM
_NSLAB_FULL = _NUM_EMBED // _LANE_PAD
_SLAB_REM = _NUM_EMBED - _NSLAB_FULL * _LANE_PAD
_FMT_ITERS = 246


def _make_format_kernel():
  mesh = plsc.VectorSubcoreMesh(core_axis_name="c", subcore_axis_name="s")

  @functools.partial(
      pl.kernel,
      mesh=mesh,
      compiler_params=pltpu.CompilerParams(
          use_tc_tiling_on_sc=True, needs_layout_passes=False),
      out_type=jax.ShapeDtypeStruct((_NUM_EMBED, _LANE_PAD), jnp.float32),
      scratch_types=[
          pltpu.VMEM((2, _EMBED_DIM, _LANE_PAD), jnp.float32),
          pltpu.VMEM((2, _LANE_PAD, _LANE_PAD), jnp.float32),
          pltpu.VMEM((_EMBED_DIM, _SLAB_REM), jnp.float32),
          pltpu.SemaphoreType.DMA((2,)),
          pltpu.SemaphoreType.DMA((2,)),
      ],
  )
  def k(wt_hbm, wrem_hbm, w128_hbm, slab_v, tslab_v, slab64_v, isem, osem):
    wid = lax.axis_index("s") * _NC + lax.axis_index("c")

    row_ids = [lax.iota(jnp.int32, 16) + 16 * q for q in range(8)]

    def fire_in(j, p):
      pltpu.async_copy(
          wt_hbm.at[:, pl.ds(j * _LANE_PAD, _LANE_PAD)],
          slab_v.at[p], isem.at[p])

    def wait_in(p):
      pltpu.make_async_copy(
          wt_hbm.at[:, pl.ds(0, _LANE_PAD)], slab_v.at[p], isem.at[p]).wait()

    def fire_out(j, p):
      pltpu.async_copy(
          tslab_v.at[p],
          w128_hbm.at[pl.ds(j * _LANE_PAD, _LANE_PAD)],
          osem.at[p])

    def wait_out(p):
      pltpu.make_async_copy(
          tslab_v.at[p],
          w128_hbm.at[pl.ds(0, _LANE_PAD)],
          osem.at[p]).wait()

    def transpose(src, dst, n_q):
      for d in range(0, _EMBED_DIM, 2):
        cols = [jnp.full((16,), d + e, jnp.int32) for e in (0, 1)]
        vs = [(e, q, src[d + e, pl.ds(16 * q, 16)])
              for e in (0, 1) for q in range(n_q)]
        for e, q, v in vs:
          plsc.store_scatter(dst, [row_ids[q], cols[e]], v)

    fire_in(wid, 0)
    fire_in(wid + _NW, 1)

    def body(m, carry):
      for p in (0, 1):
        kk = 2 * m + p
        j = wid + _NW * kk

        @pl.when(j < _NSLAB_FULL)
        def _():
          wait_in(p)

          @pl.when(kk >= 2)
          def _():
            wait_out(p)

          transpose(slab_v.at[p], tslab_v.at[p], 8)
          fire_out(j, p)
          jn = wid + _NW * (kk + 2)

          @pl.when(jn < _NSLAB_FULL)
          def _():
            fire_in(jn, p)

      return carry

    lax.fori_loop(0, _FMT_ITERS // 2, body, 0)

    wait_out(0)
    wait_out(1)

    @pl.when(wid == 0)
    def _():
      base = _NSLAB_FULL * _LANE_PAD
      pltpu.sync_copy(wrem_hbm, slab64_v)
      transpose(slab64_v, tslab_v.at[0], _SLAB_REM // 16)
      pltpu.sync_copy(
          tslab_v.at[0, pl.ds(0, _SLAB_REM)],
          w128_hbm.at[pl.ds(base, _SLAB_REM)])

  return k


def _make_gather_kernel():
  mesh = plsc.VectorSubcoreMesh(core_axis_name="c", subcore_axis_name="s")

  @functools.partial(
      pl.kernel,
      mesh=mesh,
      compiler_params=pltpu.CompilerParams(use_tc_tiling_on_sc=False),
      out_type=jax.ShapeDtypeStruct((_BATCH, _HIST, _LANE_PAD), jnp.float32),
      scratch_types=[
          pltpu.VMEM((_ROWS_W, _HIST), jnp.int32),
          pltpu.VMEM((2, _NBUF, _HIST, _LANE_PAD), jnp.float32),
          pltpu.SemaphoreType.DMA((2, _NBUF)),
          pltpu.SemaphoreType.DMA((2, _NBUF)),
      ],
  )
  def k(idx_hbm, table_hbm, out_hbm, idx_v, rows_v, gsem, ssem):
    wid = lax.axis_index("s") * _NC + lax.axis_index("c")
    base = wid * _ROWS_W

    pltpu.sync_copy(idx_hbm.at[pl.ds(base, _ROWS_W)], idx_v)

    def fire_gather(row, p, b):
      pltpu.async_copy(
          table_hbm.at[idx_v.at[row]], rows_v.at[p, b], gsem.at[p, b])

    def fire_store(row, p, b):
      pltpu.async_copy(
          rows_v.at[p, b, :, pl.ds(0, _EMBED_DIM)],
          out_hbm.at[base + row, :, pl.ds(0, _EMBED_DIM)],
          ssem.at[p, b])

    def wait_gather(p, b):
      pltpu.make_async_copy(
          table_hbm.at[idx_v.at[0]], rows_v.at[p, b], gsem.at[p, b]).wait()

    def wait_store(p, b):
      pltpu.make_async_copy(
          rows_v.at[p, b, :, pl.ds(0, _EMBED_DIM)],
          out_hbm.at[base, :, pl.ds(0, _EMBED_DIM)],
          ssem.at[p, b]).wait()

    for b in range(_NBUF):
      fire_gather(b, 0, b)

    def body(j, carry):
      for p in (0, 1):
        i = 2 * j + p
        q = 1 - p
        for b in range(_NBUF):
          @pl.when(i > 0)
          def _():
            wait_store(q, b)

          @pl.when(i < _NGROUP - 1)
          def _():
            fire_gather((i + 1) * _NBUF + b, q, b)

          wait_gather(p, b)
          fire_store(i * _NBUF + b, p, b)
      return carry

    lax.fori_loop(0, _NGROUP // 2, body, 0)

    for b in range(_NBUF):
      wait_store(1, b)

  return k


_format_call = _make_format_kernel()
_gather_call = _make_gather_kernel()


def kernel(idx, weight):
  wt = weight.T
  w128 = _format_call(wt, wt[:, _NSLAB_FULL * _LANE_PAD:])
  out_pad = _gather_call(idx.astype(jnp.int32), w128)
  return out_pad[:, :, :_EMBED_DIM]


# speedup vs baseline: 2.0900x; 2.0772x over previous
import functools

import jax
import jax.numpy as jnp
from jax import lax
from jax.experimental import pallas as pl
from jax.experimental.pallas import tpu as pltpu
from jax.experimental.pallas import tpu_sc as plsc

_NUM_EMBED = 1000000
_EMBED_DIM = 64
_BATCH = 4096
_HIST = 200

_NC = 2
_NS = 16
_NW = _NC * _NS

_ROWS_W = _BATCH // _NW
_NBUF = 2
_NGROUP = _ROWS_W // _NBUF

_LANE_PAD = 2 * _EMBED_DIM
_NSLAB_FULL = _NUM_EMBED // _LANE_PAD
_SLAB_REM = _NUM_EMBED - _NSLAB_FULL * _LANE_PAD
_FMT_ITERS = 246


def _make_format_kernel():
  mesh = plsc.VectorSubcoreMesh(core_axis_name="c", subcore_axis_name="s")

  @functools.partial(
      pl.kernel,
      mesh=mesh,
      compiler_params=pltpu.CompilerParams(
          use_tc_tiling_on_sc=True, needs_layout_passes=False),
      out_type=jax.ShapeDtypeStruct((_NUM_EMBED, _LANE_PAD), jnp.float32),
      scratch_types=[
          pltpu.VMEM((2, _EMBED_DIM, _LANE_PAD), jnp.float32),
          pltpu.VMEM((2, _LANE_PAD, _LANE_PAD), jnp.float32),
          pltpu.VMEM((_EMBED_DIM, _SLAB_REM), jnp.float32),
          pltpu.SemaphoreType.DMA((2,)),
          pltpu.SemaphoreType.DMA((2,)),
      ],
  )
  def k(wt_hbm, wrem_hbm, w128_hbm, slab_v, tslab_v, slab64_v, isem, osem):
    wid = lax.axis_index("s") * _NC + lax.axis_index("c")

    iota16 = lax.iota(jnp.int32, 16)
    rot = [(iota16 + k) & 15 for k in range(16)]

    def fire_in(j, p):
      pltpu.async_copy(
          wt_hbm.at[:, pl.ds(j * _LANE_PAD, _LANE_PAD)],
          slab_v.at[p], isem.at[p])

    def wait_in(p):
      pltpu.make_async_copy(
          wt_hbm.at[:, pl.ds(0, _LANE_PAD)], slab_v.at[p], isem.at[p]).wait()

    def fire_out(j, p):
      pltpu.async_copy(
          tslab_v.at[p],
          w128_hbm.at[pl.ds(j * _LANE_PAD, _LANE_PAD)],
          osem.at[p])

    def wait_out(p):
      pltpu.make_async_copy(
          tslab_v.at[p],
          w128_hbm.at[pl.ds(0, _LANE_PAD)],
          osem.at[p]).wait()

    def transpose(src, dst, n_q):
      def dblock(t, carry):
        dvec = iota16 + 16 * t
        for rb in range(0, 16 * n_q, 16):
          rvecs = [rot[k] + rb for k in range(16)]
          vs = [plsc.load_gather(src, [dvec, rvecs[k]]) for k in range(16)]
          for k in range(16):
            plsc.store_scatter(dst, [rvecs[k], dvec], vs[k])
        return carry

      lax.fori_loop(0, _EMBED_DIM // 16, dblock, 0)

    fire_in(wid, 0)
    fire_in(wid + _NW, 1)

    def body(m, carry):
      for p in (0, 1):
        kk = 2 * m + p
        j = wid + _NW * kk

        @pl.when(j < _NSLAB_FULL)
        def _():
          wait_in(p)

          @pl.when(kk >= 2)
          def _():
            wait_out(p)

          transpose(slab_v.at[p], tslab_v.at[p], 8)
          fire_out(j, p)
          jn = wid + _NW * (kk + 2)

          @pl.when(jn < _NSLAB_FULL)
          def _():
            fire_in(jn, p)

      return carry

    lax.fori_loop(0, _FMT_ITERS // 2, body, 0)

    wait_out(0)
    wait_out(1)

    @pl.when(wid == 0)
    def _():
      base = _NSLAB_FULL * _LANE_PAD
      pltpu.sync_copy(wrem_hbm, slab64_v)
      transpose(slab64_v, tslab_v.at[0], _SLAB_REM // 16)
      pltpu.sync_copy(
          tslab_v.at[0, pl.ds(0, _SLAB_REM)],
          w128_hbm.at[pl.ds(base, _SLAB_REM)])

  return k


def _make_gather_kernel():
  mesh = plsc.VectorSubcoreMesh(core_axis_name="c", subcore_axis_name="s")

  @functools.partial(
      pl.kernel,
      mesh=mesh,
      compiler_params=pltpu.CompilerParams(use_tc_tiling_on_sc=False),
      out_type=jax.ShapeDtypeStruct((_BATCH, _HIST, _LANE_PAD), jnp.float32),
      scratch_types=[
          pltpu.VMEM((_ROWS_W, _HIST), jnp.int32),
          pltpu.VMEM((2, _NBUF, _HIST, _LANE_PAD), jnp.float32),
          pltpu.SemaphoreType.DMA((2, _NBUF)),
          pltpu.SemaphoreType.DMA((2, _NBUF)),
      ],
  )
  def k(idx_hbm, table_hbm, out_hbm, idx_v, rows_v, gsem, ssem):
    wid = lax.axis_index("s") * _NC + lax.axis_index("c")
    base = wid * _ROWS_W

    pltpu.sync_copy(idx_hbm.at[pl.ds(base, _ROWS_W)], idx_v)

    def fire_gather(row, p, b):
      pltpu.async_copy(
          table_hbm.at[idx_v.at[row]], rows_v.at[p, b], gsem.at[p, b])

    def fire_store(row, p, b):
      pltpu.async_copy(
          rows_v.at[p, b, :, pl.ds(0, _EMBED_DIM)],
          out_hbm.at[base + row, :, pl.ds(0, _EMBED_DIM)],
          ssem.at[p, b])

    def wait_gather(p, b):
      pltpu.make_async_copy(
          table_hbm.at[idx_v.at[0]], rows_v.at[p, b], gsem.at[p, b]).wait()

    def wait_store(p, b):
      pltpu.make_async_copy(
          rows_v.at[p, b, :, pl.ds(0, _EMBED_DIM)],
          out_hbm.at[base, :, pl.ds(0, _EMBED_DIM)],
          ssem.at[p, b]).wait()

    for b in range(_NBUF):
      fire_gather(b, 0, b)

    def body(j, carry):
      for p in (0, 1):
        i = 2 * j + p
        q = 1 - p
        for b in range(_NBUF):
          @pl.when(i > 0)
          def _():
            wait_store(q, b)

          @pl.when(i < _NGROUP - 1)
          def _():
            fire_gather((i + 1) * _NBUF + b, q, b)

          wait_gather(p, b)
          fire_store(i * _NBUF + b, p, b)
      return carry

    lax.fori_loop(0, _NGROUP // 2, body, 0)

    for b in range(_NBUF):
      wait_store(1, b)

  return k


_format_call = _make_format_kernel()
_gather_call = _make_gather_kernel()


def kernel(idx, weight):
  wt = weight.T
  w128 = _format_call(wt, wt[:, _NSLAB_FULL * _LANE_PAD:])
  out_pad = _gather_call(idx.astype(jnp.int32), w128)
  return out_pad[:, :, :_EMBED_DIM]


# speedup vs baseline: 2.4981x; 1.1953x over previous
import functools

import jax
import jax.numpy as jnp
from jax import lax
from jax.experimental import pallas as pl
from jax.experimental.pallas import tpu as pltpu
from jax.experimental.pallas import tpu_sc as plsc

_NUM_EMBED = 1000000
_EMBED_DIM = 64
_BATCH = 4096
_HIST = 200

_NC = 2
_NS = 16
_NW = _NC * _NS

_ROWS_W = _BATCH // _NW
_NBUF = 2
_NGROUP = _ROWS_W // _NBUF

_LANE_PAD = 2 * _EMBED_DIM
_NSLAB_FULL = _NUM_EMBED // _LANE_PAD
_SLAB_REM = _NUM_EMBED - _NSLAB_FULL * _LANE_PAD
_FMT_ITERS = 246


def _make_format_kernel():
  mesh = plsc.VectorSubcoreMesh(core_axis_name="c", subcore_axis_name="s")

  @functools.partial(
      pl.kernel,
      mesh=mesh,
      compiler_params=pltpu.CompilerParams(
          use_tc_tiling_on_sc=True, needs_layout_passes=False),
      out_type=jax.ShapeDtypeStruct((_NUM_EMBED // 2, _LANE_PAD),
                                    jnp.float32),
      scratch_types=[
          pltpu.VMEM((2, _EMBED_DIM, _LANE_PAD), jnp.float32),
          pltpu.VMEM((2, _EMBED_DIM, _LANE_PAD), jnp.float32),
          pltpu.VMEM((_EMBED_DIM, _SLAB_REM), jnp.float32),
          pltpu.SemaphoreType.DMA((2,)),
          pltpu.SemaphoreType.DMA((2,)),
      ],
  )
  def k(wt_hbm, wrem_hbm, w128_hbm, slab_v, tslab_v, slab64_v, isem, osem):
    wid = lax.axis_index("s") * _NC + lax.axis_index("c")

    iota16 = lax.iota(jnp.int32, 16)
    rot = [(iota16 + k) & 15 for k in range(16)]
    rot_half = [r >> 1 for r in rot]
    rot_col = [(r & 1) << 6 for r in rot]

    def fire_in(j, p):
      pltpu.async_copy(
          wt_hbm.at[:, pl.ds(j * _LANE_PAD, _LANE_PAD)],
          slab_v.at[p], isem.at[p])

    def wait_in(p):
      pltpu.make_async_copy(
          wt_hbm.at[:, pl.ds(0, _LANE_PAD)], slab_v.at[p], isem.at[p]).wait()

    def fire_out(j, p):
      pltpu.async_copy(
          tslab_v.at[p],
          w128_hbm.at[pl.ds(j * _EMBED_DIM, _EMBED_DIM)],
          osem.at[p])

    def wait_out(p):
      pltpu.make_async_copy(
          tslab_v.at[p],
          w128_hbm.at[pl.ds(0, _EMBED_DIM)],
          osem.at[p]).wait()

    def transpose(src, dst, n_q):
      def dblock(t, carry):
        dvec = iota16 + 16 * t
        for rb in range(0, 16 * n_q, 16):
          rvecs = [rot[k] + rb for k in range(16)]
          vs = [plsc.load_gather(src, [dvec, rvecs[k]]) for k in range(16)]
          for k in range(16):
            plsc.store_scatter(
                dst, [rot_half[k] + rb // 2, dvec + rot_col[k]], vs[k])
        return carry

      lax.fori_loop(0, _EMBED_DIM // 16, dblock, 0)

    fire_in(wid, 0)
    fire_in(wid + _NW, 1)

    def body(m, carry):
      for p in (0, 1):
        kk = 2 * m + p
        j = wid + _NW * kk

        @pl.when(j < _NSLAB_FULL)
        def _():
          wait_in(p)

          @pl.when(kk >= 2)
          def _():
            wait_out(p)

          transpose(slab_v.at[p], tslab_v.at[p], 8)
          fire_out(j, p)
          jn = wid + _NW * (kk + 2)

          @pl.when(jn < _NSLAB_FULL)
          def _():
            fire_in(jn, p)

      return carry

    lax.fori_loop(0, _FMT_ITERS // 2, body, 0)

    wait_out(0)
    wait_out(1)

    @pl.when(wid == 0)
    def _():
      pltpu.sync_copy(wrem_hbm, slab64_v)
      transpose(slab64_v, tslab_v.at[0], _SLAB_REM // 16)
      pltpu.sync_copy(
          tslab_v.at[0, pl.ds(0, _SLAB_REM // 2)],
          w128_hbm.at[pl.ds(_NSLAB_FULL * _EMBED_DIM, _SLAB_REM // 2)])

  return k


def _make_gather_kernel():
  mesh = plsc.VectorSubcoreMesh(core_axis_name="c", subcore_axis_name="s")

  @functools.partial(
      pl.kernel,
      mesh=mesh,
      compiler_params=pltpu.CompilerParams(use_tc_tiling_on_sc=False),
      out_type=jax.ShapeDtypeStruct((_BATCH, _HIST, _LANE_PAD), jnp.float32),
      scratch_types=[
          pltpu.VMEM((_ROWS_W, _HIST), jnp.int32),
          pltpu.VMEM((2, _NBUF, _HIST, _EMBED_DIM), jnp.float32),
          pltpu.SemaphoreType.DMA((2, _NBUF)),
          pltpu.SemaphoreType.DMA((2, _NBUF)),
      ],
  )
  def k(idx_hbm, table_hbm, out_hbm, idx_v, rows_v, gsem, ssem):
    wid = lax.axis_index("s") * _NC + lax.axis_index("c")
    base = wid * _ROWS_W

    pltpu.sync_copy(idx_hbm.at[pl.ds(base, _ROWS_W)], idx_v)

    def fire_gather(row, p, b):
      pltpu.async_copy(
          table_hbm.at[idx_v.at[row]], rows_v.at[p, b], gsem.at[p, b])

    def fire_store(row, p, b):
      pltpu.async_copy(
          rows_v.at[p, b],
          out_hbm.at[base + row, :, pl.ds(0, _EMBED_DIM)],
          ssem.at[p, b])

    def wait_gather(p, b):
      pltpu.make_async_copy(
          table_hbm.at[idx_v.at[0]], rows_v.at[p, b], gsem.at[p, b]).wait()

    def wait_store(p, b):
      pltpu.make_async_copy(
          rows_v.at[p, b],
          out_hbm.at[base, :, pl.ds(0, _EMBED_DIM)],
          ssem.at[p, b]).wait()

    for b in range(_NBUF):
      fire_gather(b, 0, b)

    def body(j, carry):
      for p in (0, 1):
        i = 2 * j + p
        q = 1 - p
        for b in range(_NBUF):
          @pl.when(i > 0)
          def _():
            wait_store(q, b)

          @pl.when(i < _NGROUP - 1)
          def _():
            fire_gather((i + 1) * _NBUF + b, q, b)

          wait_gather(p, b)
          fire_store(i * _NBUF + b, p, b)
      return carry

    lax.fori_loop(0, _NGROUP // 2, body, 0)

    for b in range(_NBUF):
      wait_store(1, b)

  return k


_format_call = _make_format_kernel()
_gather_call = _make_gather_kernel()


def kernel(idx, weight):
  wt = weight.T
  w_packed = _format_call(wt, wt[:, _NSLAB_FULL * _LANE_PAD:])
  w64 = w_packed.reshape(_NUM_EMBED, _EMBED_DIM)
  out_pad = _gather_call(idx.astype(jnp.int32), w64)
  return out_pad[:, :, :_EMBED_DIM]
